# Initial kernel scaffold; baseline (speedup 1.0000x reference)
import jax
import jax.numpy as jnp
from jax.experimental import pallas as pl


def kernel(x, W, router_bias):
    raise NotImplementedError("write your pallas kernel here")


# speedup vs baseline: 1.3073x; 1.3073x over previous
import functools

import jax
import jax.numpy as jnp
from jax.experimental import pallas as pl

_TOP_K = 8


def _router_body(x_ref, wt_ref, b_ref, idx_ref, w_ref, p_ref, load_ref, *, tm, e):
    logits = jnp.dot(x_ref[...], wt_ref[...], preferred_element_type=jnp.float32)

    m = jnp.max(logits, axis=-1, keepdims=True)
    ex = jnp.exp(logits - m)
    probs = ex / jnp.sum(ex, axis=-1, keepdims=True)
    p_ref[...] = probs

    cols = jax.lax.broadcasted_iota(jnp.int32, (tm, e), 1)
    work = logits + b_ref[...]
    sel = jnp.zeros((tm, e), dtype=jnp.bool_)
    idx_cols = []
    w_cols = []
    neg_inf = jnp.float32(-jnp.inf)
    for _ in range(_TOP_K):
        mk = jnp.max(work, axis=-1, keepdims=True)
        idx = jnp.min(jnp.where(work == mk, cols, e), axis=-1, keepdims=True)
        onehot = cols == idx
        idx_cols.append(idx)
        w_cols.append(jnp.sum(jnp.where(onehot, probs, 0.0), axis=-1, keepdims=True))
        sel = sel | onehot
        work = jnp.where(onehot, neg_inf, work)

    indices = jnp.concatenate(idx_cols, axis=-1)
    weights = jnp.concatenate(w_cols, axis=-1)
    weights = weights / jnp.clip(jnp.sum(weights, axis=-1, keepdims=True), 1e-9, None)
    idx_ref[...] = indices
    w_ref[...] = weights

    @pl.when(pl.program_id(0) == 0)
    def _():
        load_ref[...] = jnp.zeros_like(load_ref)

    load_ref[...] += jnp.sum(sel.astype(jnp.float32), axis=0, keepdims=True)


def kernel(x, W, router_bias):
    tokens, dim = x.shape
    e = W.shape[0]
    tm = min(512, tokens)
    grid = tokens // tm

    wt = W.T
    bias2d = router_bias.reshape(1, e)

    out_shapes = (
        jax.ShapeDtypeStruct((tokens, _TOP_K), jnp.int32),
        jax.ShapeDtypeStruct((tokens, _TOP_K), jnp.float32),
        jax.ShapeDtypeStruct((tokens, e), jnp.float32),
        jax.ShapeDtypeStruct((1, e), jnp.float32),
    )
    indices, weights, probs, load = pl.pallas_call(
        functools.partial(_router_body, tm=tm, e=e),
        grid=(grid,),
        in_specs=[
            pl.BlockSpec((tm, dim), lambda i: (i, 0)),
            pl.BlockSpec((dim, e), lambda i: (0, 0)),
            pl.BlockSpec((1, e), lambda i: (0, 0)),
        ],
        out_specs=(
            pl.BlockSpec((tm, _TOP_K), lambda i: (i, 0)),
            pl.BlockSpec((tm, _TOP_K), lambda i: (i, 0)),
            pl.BlockSpec((tm, e), lambda i: (i, 0)),
            pl.BlockSpec((1, e), lambda i: (0, 0)),
        ),
        out_shape=out_shapes,
    )(x, wt, bias2d)

    return indices, weights.astype(x.dtype), probs, load.reshape(e)


# speedup vs baseline: 1.4699x; 1.1244x over previous
import functools

import jax
import jax.numpy as jnp
from jax.experimental import pallas as pl

_TOP_K = 8


def _router_body(x_ref, wt_ref, b_ref, idx_ref, w_ref, p_ref, load_ref, *, tm, e):
    logits = jnp.dot(x_ref[...], wt_ref[...], preferred_element_type=jnp.float32)

    m = jnp.max(logits, axis=-1, keepdims=True)
    ex = jnp.exp(logits - m)
    probs = ex / jnp.sum(ex, axis=-1, keepdims=True)
    p_ref[...] = probs

    work = logits + b_ref[...]
    i = jax.lax.bitcast_convert_type(work, jnp.int32)
    key = i ^ ((i >> 31) & jnp.int32(0x7FFFFFFF))
    cols = jax.lax.broadcasted_iota(jnp.int32, (tm, e), 1)
    key = (key & jnp.int32(~0x3F)) | (jnp.int32(e - 1) - cols)

    sel = jnp.zeros((tm, e), dtype=jnp.bool_)
    idx_cols = []
    key_cols = []
    int_min = jnp.int32(-2147483648)
    for _ in range(_TOP_K):
        mk = jnp.max(key, axis=-1, keepdims=True)
        onehot = key == mk
        idx_cols.append(jnp.int32(e - 1) - (mk & jnp.int32(0x3F)))
        key_cols.append(mk)
        sel = sel | onehot
        key = jnp.where(onehot, int_min, key)

    indices = jnp.concatenate(idx_cols, axis=-1)
    topk_keys = jnp.concatenate(key_cols, axis=-1)
    vi = topk_keys | jnp.int32(0x3F)
    vi = vi ^ ((vi >> 31) & jnp.int32(0x7FFFFFFF))
    vals = jax.lax.bitcast_convert_type(vi, jnp.float32)
    ew = jnp.exp(vals - m)
    weights = ew / jnp.clip(jnp.sum(ew, axis=-1, keepdims=True), 1e-9, None)
    idx_ref[...] = indices
    w_ref[...] = weights

    @pl.when(pl.program_id(0) == 0)
    def _():
        load_ref[...] = jnp.zeros_like(load_ref)

    load_ref[...] += jnp.sum(sel.astype(jnp.float32), axis=0, keepdims=True)


def kernel(x, W, router_bias):
    tokens, dim = x.shape
    e = W.shape[0]
    tm = min(512, tokens)
    grid = tokens // tm

    wt = W.T
    bias2d = router_bias.reshape(1, e)

    out_shapes = (
        jax.ShapeDtypeStruct((tokens, _TOP_K), jnp.int32),
        jax.ShapeDtypeStruct((tokens, _TOP_K), jnp.float32),
        jax.ShapeDtypeStruct((tokens, e), jnp.float32),
        jax.ShapeDtypeStruct((1, e), jnp.float32),
    )
    indices, weights, probs, load = pl.pallas_call(
        functools.partial(_router_body, tm=tm, e=e),
        grid=(grid,),
        in_specs=[
            pl.BlockSpec((tm, dim), lambda i: (i, 0)),
            pl.BlockSpec((dim, e), lambda i: (0, 0)),
            pl.BlockSpec((1, e), lambda i: (0, 0)),
        ],
        out_specs=(
            pl.BlockSpec((tm, _TOP_K), lambda i: (i, 0)),
            pl.BlockSpec((tm, _TOP_K), lambda i: (i, 0)),
            pl.BlockSpec((tm, e), lambda i: (i, 0)),
            pl.BlockSpec((1, e), lambda i: (0, 0)),
        ),
        out_shape=out_shapes,
    )(x, wt, bias2d)

    return indices, weights.astype(x.dtype), probs, load.reshape(e)


# speedup vs baseline: 1.6219x; 1.1034x over previous
import functools

import jax
import jax.numpy as jnp
from jax.experimental import pallas as pl

_TOP_K = 8


def _router_body(x_ref, wt_ref, b_ref, idx_ref, w_ref, p_ref, load_ref, *, tm, e):
    logits = jnp.dot(x_ref[...], wt_ref[...], preferred_element_type=jnp.float32)

    m = jnp.max(logits, axis=-1, keepdims=True)
    ex = jnp.exp(logits - m)
    probs = ex / jnp.sum(ex, axis=-1, keepdims=True)
    p_ref[...] = probs

    work = logits + b_ref[...]
    i = jax.lax.bitcast_convert_type(work, jnp.int32)
    key = i ^ ((i >> 31) & jnp.int32(0x7FFFFFFF))
    cols = jax.lax.broadcasted_iota(jnp.int32, (tm, e), 1)
    key = (key & jnp.int32(~0x3F)) | (jnp.int32(e - 1) - cols)

    sel = jnp.zeros((tm, e), dtype=jnp.bool_)
    idx_cols = []
    key_cols = []
    int_min = jnp.int32(-2147483648)
    for _ in range(_TOP_K):
        mk = jnp.max(key, axis=-1, keepdims=True)
        onehot = key == mk
        idx_cols.append(jnp.int32(e - 1) - (mk & jnp.int32(0x3F)))
        key_cols.append(mk)
        sel = sel | onehot
        key = jnp.where(onehot, int_min, key)

    indices = jnp.concatenate(idx_cols, axis=-1)
    topk_keys = jnp.concatenate(key_cols, axis=-1)
    vi = topk_keys | jnp.int32(0x3F)
    vi = vi ^ ((vi >> 31) & jnp.int32(0x7FFFFFFF))
    vals = jax.lax.bitcast_convert_type(vi, jnp.float32)
    ew = jnp.exp(vals - m)
    weights = ew / jnp.clip(jnp.sum(ew, axis=-1, keepdims=True), 1e-9, None)
    idx_ref[...] = indices
    w_ref[...] = weights

    @pl.when(pl.program_id(0) == 0)
    def _():
        load_ref[...] = jnp.zeros_like(load_ref)

    load_ref[...] += jnp.sum(sel.astype(jnp.float32), axis=0, keepdims=True)


def kernel(x, W, router_bias):
    tokens, dim = x.shape
    e = W.shape[0]
    tm = min(1024, tokens)
    grid = tokens // tm

    wt = W.T
    bias2d = router_bias.reshape(1, e)

    out_shapes = (
        jax.ShapeDtypeStruct((tokens, _TOP_K), jnp.int32),
        jax.ShapeDtypeStruct((tokens, _TOP_K), jnp.float32),
        jax.ShapeDtypeStruct((tokens, e), jnp.float32),
        jax.ShapeDtypeStruct((1, e), jnp.float32),
    )
    indices, weights, probs, load = pl.pallas_call(
        functools.partial(_router_body, tm=tm, e=e),
        grid=(grid,),
        in_specs=[
            pl.BlockSpec((tm, dim), lambda i: (i, 0)),
            pl.BlockSpec((dim, e), lambda i: (0, 0)),
            pl.BlockSpec((1, e), lambda i: (0, 0)),
        ],
        out_specs=(
            pl.BlockSpec((tm, _TOP_K), lambda i: (i, 0)),
            pl.BlockSpec((tm, _TOP_K), lambda i: (i, 0)),
            pl.BlockSpec((tm, e), lambda i: (i, 0)),
            pl.BlockSpec((1, e), lambda i: (0, 0)),
        ),
        out_shape=out_shapes,
    )(x, wt, bias2d)

    return indices, weights.astype(x.dtype), probs, load.reshape(e)


# speedup vs baseline: 1.7122x; 1.0557x over previous
import functools

import jax
import jax.numpy as jnp
from jax import lax
from jax.experimental import pallas as pl
from jax.experimental.pallas import tpu as pltpu

_TOP_K = 8


def _router_body(x_ref, wt_ref, b_ref, idx_ref, w_ref, p_ref, load_ref, lbuf,
                 *, tm, e, n):
    i = pl.program_id(0)
    slot = lax.rem(i, 2)

    lbuf[slot] = jnp.dot(x_ref[...], wt_ref[...],
                         preferred_element_type=jnp.float32)

    logits = lbuf[1 - slot]

    m = jnp.max(logits, axis=-1, keepdims=True)
    ex = jnp.exp(logits - m)
    probs = ex / jnp.sum(ex, axis=-1, keepdims=True)
    p_ref[...] = probs

    work = logits + b_ref[...]
    ki = lax.bitcast_convert_type(work, jnp.int32)
    ki = ki ^ ((ki >> 31) & jnp.int32(0x7FFFFFFF))
    cols = lax.broadcasted_iota(jnp.int32, (tm, e), 1)
    ki = (ki & jnp.int32(~0x3F)) | (jnp.int32(e - 1) - cols)
    ki = ki ^ ((ki >> 31) & jnp.int32(0x7FFFFFFF))
    key = lax.bitcast_convert_type(ki, jnp.float32)

    sel = jnp.zeros((tm, e), dtype=jnp.bool_)
    key_cols = []
    neg_inf = jnp.float32(-jnp.inf)
    for _ in range(_TOP_K):
        mk = jnp.max(key, axis=-1, keepdims=True)
        onehot = key == mk
        key_cols.append(mk)
        sel = sel | onehot
        key = jnp.where(onehot, neg_inf, key)

    topk = jnp.concatenate(key_cols, axis=-1)
    tki = lax.bitcast_convert_type(topk, jnp.int32)
    low = tki & jnp.int32(0x3F)
    indices = jnp.where(tki < 0, low, jnp.int32(e - 1) - low)
    ew = jnp.exp(topk - m)
    weights = ew / jnp.clip(jnp.sum(ew, axis=-1, keepdims=True), 1e-9, None)
    idx_ref[...] = indices
    w_ref[...] = weights

    @pl.when(i == 1)
    def _():
        load_ref[...] = jnp.zeros_like(load_ref)

    @pl.when(i >= 1)
    def _():
        load_ref[...] += jnp.sum(sel.astype(jnp.float32), axis=0, keepdims=True)


def kernel(x, W, router_bias):
    tokens, dim = x.shape
    e = W.shape[0]
    tm = min(1024, tokens)
    n = tokens // tm

    wt = W.T
    bias2d = router_bias.reshape(1, e)

    out_shapes = (
        jax.ShapeDtypeStruct((tokens, _TOP_K), jnp.int32),
        jax.ShapeDtypeStruct((tokens, _TOP_K), jnp.float32),
        jax.ShapeDtypeStruct((tokens, e), jnp.float32),
        jax.ShapeDtypeStruct((1, e), jnp.float32),
    )
    indices, weights, probs, load = pl.pallas_call(
        functools.partial(_router_body, tm=tm, e=e, n=n),
        grid=(n + 1,),
        in_specs=[
            pl.BlockSpec((tm, dim), lambda i: (jnp.minimum(i, n - 1), 0)),
            pl.BlockSpec((dim, e), lambda i: (0, 0)),
            pl.BlockSpec((1, e), lambda i: (0, 0)),
        ],
        out_specs=(
            pl.BlockSpec((tm, _TOP_K), lambda i: (jnp.maximum(i - 1, 0), 0)),
            pl.BlockSpec((tm, _TOP_K), lambda i: (jnp.maximum(i - 1, 0), 0)),
            pl.BlockSpec((tm, e), lambda i: (jnp.maximum(i - 1, 0), 0)),
            pl.BlockSpec((1, e), lambda i: (0, 0)),
        ),
        out_shape=out_shapes,
        scratch_shapes=[pltpu.VMEM((2, tm, e), jnp.float32)],
    )(x, wt, bias2d)

    return indices, weights.astype(x.dtype), probs, load.reshape(e)
